# baseline (speedup 1.0000x reference)
import functools

import jax
import jax.numpy as jnp
from jax import lax
from jax.experimental import pallas as pl
from jax.experimental.pallas import tpu as pltpu
from jax.experimental.pallas import tpu_sc as plsc

N = 64
D = 16384
E = 16384
PER_COL = 16

NC = 2
NS = 16
NW = NC * NS

NP = 128
CHUNK_COLS = 16
ROWS_PER_CHUNK = CHUNK_COLS * PER_COL
IDX_BLK = 128
GPC = ROWS_PER_CHUNK // IDX_BLK

LANE = 16
NLB = N // LANE


def _tree_max(vs):
    while len(vs) > 1:
        vs = [jnp.maximum(vs[i], vs[i + 1]) for i in range(0, len(vs) - 1, 2)] + (
            [vs[-1]] if len(vs) % 2 else [])
    return vs[0]


def _tree_sum(vs):
    while len(vs) > 1:
        vs = [vs[i] + vs[i + 1] for i in range(0, len(vs) - 1, 2)] + (
            [vs[-1]] if len(vs) % 2 else [])
    return vs[0]


def _make_sc(n_shard, shard):
    e_shard = E // n_shard
    cols_per_tile = e_shard // NW
    nnz_per_tile = cols_per_tile * PER_COL
    nchunk = cols_per_tile // CHUNK_COLS
    npair = nchunk // 2

    @functools.partial(
        pl.kernel,
        out_type=jax.ShapeDtypeStruct((e_shard * NP,), jnp.float32),
        mesh=plsc.VectorSubcoreMesh(core_axis_name="c", subcore_axis_name="s"),
        scratch_types=[
            pltpu.VMEM((nnz_per_tile,), jnp.int32),
            pltpu.VMEM((nnz_per_tile,), jnp.float32),
            pltpu.VMEM((ROWS_PER_CHUNK, N), jnp.float32),
            pltpu.VMEM((ROWS_PER_CHUNK, N), jnp.float32),
            pltpu.VMEM((CHUNK_COLS * NP,), jnp.float32),
            pltpu.VMEM((CHUNK_COLS * NP,), jnp.float32),
            pltpu.SemaphoreType.DMA,
            pltpu.SemaphoreType.DMA,
            pltpu.SemaphoreType.DMA,
            pltpu.SemaphoreType.DMA,
        ],
        compiler_params=pltpu.CompilerParams(use_tc_tiling_on_sc=False),
    )
    def sc_body(xT, vals, ridx, out_o,
                idx_v, val_v, rows_a, rows_b, ob_a, ob_b,
                sem_a, sem_b, sem_oa, sem_ob):
        wid = lax.axis_index("s") * NC + lax.axis_index("c")
        nnz_base = shard * e_shard * PER_COL + wid * nnz_per_tile
        col_base = wid * cols_per_tile

        c1 = pltpu.async_copy(ridx.at[pl.ds(nnz_base, nnz_per_tile)], idx_v, sem_a)
        c2 = pltpu.async_copy(vals.at[pl.ds(nnz_base, nnz_per_tile)], val_v, sem_a)
        c1.wait()
        c2.wait()

        def issue_gather(g, buf, sem):
            for i in range(GPC):
                off = g * ROWS_PER_CHUNK + i * IDX_BLK
                pltpu.async_copy(
                    xT.at[idx_v.at[pl.ds(off, IDX_BLK)]],
                    buf.at[pl.ds(i * IDX_BLK, IDX_BLK)],
                    sem,
                )

        def wait_gather(buf, sem):
            for i in range(GPC):
                pltpu.make_async_copy(
                    xT.at[idx_v.at[pl.ds(i * IDX_BLK, IDX_BLK)]],
                    buf.at[pl.ds(i * IDX_BLK, IDX_BLK)],
                    sem,
                ).wait()

        def compute_chunk(g, buf, ob):
            @plsc.parallel_loop(0, CHUNK_COLS, unroll=2)
            def col_body(c):
                j0 = c * PER_COL
                jv = g * ROWS_PER_CHUNK + c * PER_COL
                vv = val_v[pl.ds(jv, PER_COL)]
                for l in range(NLB):
                    lanes = pl.ds(l * LANE, LANE)
                    xv = [buf[j0 + k, lanes] + vv[k] for k in range(PER_COL)]
                    m = _tree_max(xv)
                    s = _tree_sum([jnp.exp(v - m) for v in xv])
                    ob[pl.ds(c * NP + l * LANE, LANE)] = m
                    ob[pl.ds(c * NP + N + l * LANE, LANE)] = s

        def issue_out(g, ob, sem):
            dst = pl.ds((col_base + g * CHUNK_COLS) * NP, CHUNK_COLS * NP)
            pltpu.async_copy(ob, out_o.at[dst], sem)

        def wait_out(ob, sem):
            pltpu.make_async_copy(
                ob, out_o.at[pl.ds(0, CHUNK_COLS * NP)], sem).wait()

        issue_gather(0, rows_a, sem_a)

        def pair_body(p, carry):
            g0 = p * 2
            issue_gather(g0 + 1, rows_b, sem_b)
            wait_gather(rows_a, sem_a)

            @pl.when(p > 0)
            def _():
                wait_out(ob_a, sem_oa)

            compute_chunk(g0, rows_a, ob_a)
            issue_out(g0, ob_a, sem_oa)

            @pl.when(p + 1 < npair)
            def _():
                issue_gather(g0 + 2, rows_a, sem_a)

            wait_gather(rows_b, sem_b)

            @pl.when(p > 0)
            def _():
                wait_out(ob_b, sem_ob)

            compute_chunk(g0 + 1, rows_b, ob_b)
            issue_out(g0 + 1, ob_b, sem_ob)
            return carry

        lax.fori_loop(0, npair, pair_body, 0)
        wait_out(ob_a, sem_oa)
        wait_out(ob_b, sem_ob)

    return sc_body


_N_SHARD = 1
_sc_shards = [_make_sc(_N_SHARD, s) for s in range(_N_SHARD)]


_EB = 2048


def _finish_body(packed_ref, o_ref):
    mb = packed_ref[...].reshape(_EB, NP)
    o_ref[...] = (mb[:, :N] + jnp.log(mb[:, N:])).T


def _make_finish(e_shard):
    return pl.pallas_call(
        _finish_body,
        grid=(e_shard // _EB,),
        in_specs=[pl.BlockSpec((_EB * NP,), lambda i: (i,))],
        out_specs=pl.BlockSpec((N, _EB), lambda i: (0, i)),
        out_shape=jax.ShapeDtypeStruct((N, e_shard), jnp.float32),
    )


_finish = _make_finish(E // _N_SHARD)


def kernel(x, A_values, A_row_idx, A_col_idx):
    del A_col_idx
    xT = x.T
    ridx = A_row_idx.astype(jnp.int32)
    parts = []
    for sc in _sc_shards:
        packed = sc(xT, A_values, ridx)
        parts.append(_finish(packed))
    return jnp.concatenate(parts, axis=1) if len(parts) > 1 else parts[0]


# speedup vs baseline: 1.0762x; 1.0762x over previous
import functools

import jax
import jax.numpy as jnp
from jax import lax
from jax.experimental import pallas as pl
from jax.experimental.pallas import tpu as pltpu
from jax.experimental.pallas import tpu_sc as plsc

N = 64
D = 16384
E = 16384
PER_COL = 16

NC = 2
NS = 16
NW = NC * NS

NP = 128
CHUNK_COLS = 16
ROWS_PER_CHUNK = CHUNK_COLS * PER_COL
IDX_BLK = 128
GPC = ROWS_PER_CHUNK // IDX_BLK

LANE = 16
NLB = N // LANE


def _tree_max(vs):
    while len(vs) > 1:
        vs = [jnp.maximum(vs[i], vs[i + 1]) for i in range(0, len(vs) - 1, 2)] + (
            [vs[-1]] if len(vs) % 2 else [])
    return vs[0]


def _tree_sum(vs):
    while len(vs) > 1:
        vs = [vs[i] + vs[i + 1] for i in range(0, len(vs) - 1, 2)] + (
            [vs[-1]] if len(vs) % 2 else [])
    return vs[0]


def _make_sc(n_shard, shard):
    e_shard = E // n_shard
    cols_per_tile = e_shard // NW
    nnz_per_tile = cols_per_tile * PER_COL
    nchunk = cols_per_tile // CHUNK_COLS
    npair = nchunk // 2

    @functools.partial(
        pl.kernel,
        out_type=jax.ShapeDtypeStruct((e_shard * NP,), jnp.float32),
        mesh=plsc.VectorSubcoreMesh(core_axis_name="c", subcore_axis_name="s"),
        scratch_types=[
            pltpu.VMEM((nnz_per_tile,), jnp.int32),
            pltpu.VMEM((nnz_per_tile,), jnp.float32),
            pltpu.VMEM((ROWS_PER_CHUNK, NP), jnp.float32),
            pltpu.VMEM((ROWS_PER_CHUNK, NP), jnp.float32),
            pltpu.VMEM((CHUNK_COLS * NP,), jnp.float32),
            pltpu.VMEM((CHUNK_COLS * NP,), jnp.float32),
            pltpu.SemaphoreType.DMA,
            pltpu.SemaphoreType.DMA,
            pltpu.SemaphoreType.DMA,
            pltpu.SemaphoreType.DMA,
        ],
    )
    def sc_body(xT, vals, ridx, out_o,
                idx_v, val_v, rows_a, rows_b, ob_a, ob_b,
                sem_a, sem_b, sem_oa, sem_ob):
        wid = lax.axis_index("s") * NC + lax.axis_index("c")
        nnz_base = shard * e_shard * PER_COL + wid * nnz_per_tile
        col_base = wid * cols_per_tile

        c1 = pltpu.async_copy(ridx.at[pl.ds(nnz_base, nnz_per_tile)], idx_v, sem_a)
        c2 = pltpu.async_copy(vals.at[pl.ds(nnz_base, nnz_per_tile)], val_v, sem_a)
        c1.wait()
        c2.wait()

        def issue_gather(g, buf, sem):
            for i in range(GPC):
                off = g * ROWS_PER_CHUNK + i * IDX_BLK
                pltpu.async_copy(
                    xT.at[idx_v.at[pl.ds(off, IDX_BLK)]],
                    buf.at[pl.ds(i * IDX_BLK, IDX_BLK)],
                    sem,
                )

        def wait_gather(buf, sem):
            for i in range(GPC):
                pltpu.make_async_copy(
                    xT.at[idx_v.at[pl.ds(i * IDX_BLK, IDX_BLK)]],
                    buf.at[pl.ds(i * IDX_BLK, IDX_BLK)],
                    sem,
                ).wait()

        def compute_chunk(g, buf, ob):
            @plsc.parallel_loop(0, CHUNK_COLS, unroll=1)
            def col_body(c):
                j0 = c * PER_COL
                jv = g * ROWS_PER_CHUNK + c * PER_COL
                vv = val_v[pl.ds(jv, PER_COL)]
                for l in range(NLB):
                    lanes = pl.ds(l * LANE, LANE)
                    xv = [buf[j0 + k, lanes] + vv[k] for k in range(PER_COL)]
                    m = _tree_max(xv)
                    s = _tree_sum([jnp.exp(v - m) for v in xv])
                    ob[pl.ds(c * NP + l * LANE, LANE)] = m
                    ob[pl.ds(c * NP + N + l * LANE, LANE)] = s

        def issue_out(g, ob, sem):
            dst = pl.ds((col_base + g * CHUNK_COLS) * NP, CHUNK_COLS * NP)
            pltpu.async_copy(ob, out_o.at[dst], sem)

        def wait_out(ob, sem):
            pltpu.make_async_copy(
                ob, out_o.at[pl.ds(0, CHUNK_COLS * NP)], sem).wait()

        issue_gather(0, rows_a, sem_a)

        def pair_body(p, carry):
            g0 = p * 2
            issue_gather(g0 + 1, rows_b, sem_b)
            wait_gather(rows_a, sem_a)

            @pl.when(p > 0)
            def _():
                wait_out(ob_a, sem_oa)

            compute_chunk(g0, rows_a, ob_a)
            issue_out(g0, ob_a, sem_oa)

            @pl.when(p + 1 < npair)
            def _():
                issue_gather(g0 + 2, rows_a, sem_a)

            wait_gather(rows_b, sem_b)

            @pl.when(p > 0)
            def _():
                wait_out(ob_b, sem_ob)

            compute_chunk(g0 + 1, rows_b, ob_b)
            issue_out(g0 + 1, ob_b, sem_ob)
            return carry

        lax.fori_loop(0, npair, pair_body, 0)
        wait_out(ob_a, sem_oa)
        wait_out(ob_b, sem_ob)

    return sc_body


_N_SHARD = 1
_sc_shards = [_make_sc(_N_SHARD, s) for s in range(_N_SHARD)]


_EB = 2048


def _finish_body(packed_ref, o_ref):
    mb = packed_ref[...].reshape(_EB, NP)
    o_ref[...] = (mb[:, :N] + jnp.log(mb[:, N:])).T


def _make_finish(e_shard):
    return pl.pallas_call(
        _finish_body,
        grid=(e_shard // _EB,),
        in_specs=[pl.BlockSpec((_EB * NP,), lambda i: (i,))],
        out_specs=pl.BlockSpec((N, _EB), lambda i: (0, i)),
        out_shape=jax.ShapeDtypeStruct((N, e_shard), jnp.float32),
    )


_finish = _make_finish(E // _N_SHARD)


def kernel(x, A_values, A_row_idx, A_col_idx):
    del A_col_idx
    xT = jnp.concatenate([x.T, jnp.zeros((D, NP - N), jnp.float32)], axis=1)
    ridx = A_row_idx.astype(jnp.int32)
    parts = []
    for sc in _sc_shards:
        packed = sc(xT, A_values, ridx)
        parts.append(_finish(packed))
    return jnp.concatenate(parts, axis=1) if len(parts) > 1 else parts[0]


# speedup vs baseline: 1.1814x; 1.0978x over previous
import functools

import jax
import jax.numpy as jnp
from jax import lax
from jax.experimental import pallas as pl
from jax.experimental.pallas import tpu as pltpu
from jax.experimental.pallas import tpu_sc as plsc

N = 64
D = 16384
E = 16384
PER_COL = 16

NC = 2
NS = 16
NW = NC * NS

NP = 128
CHUNK_COLS = 16
ROWS_PER_CHUNK = CHUNK_COLS * PER_COL
IDX_BLK = 128
GPC = ROWS_PER_CHUNK // IDX_BLK

LANE = 16
NLB = N // LANE


def _tree_max(vs):
    while len(vs) > 1:
        vs = [jnp.maximum(vs[i], vs[i + 1]) for i in range(0, len(vs) - 1, 2)] + (
            [vs[-1]] if len(vs) % 2 else [])
    return vs[0]


def _tree_sum(vs):
    while len(vs) > 1:
        vs = [vs[i] + vs[i + 1] for i in range(0, len(vs) - 1, 2)] + (
            [vs[-1]] if len(vs) % 2 else [])
    return vs[0]


def _make_sc(n_shard, shard):
    e_shard = E // n_shard
    cols_per_tile = e_shard // NW
    nnz_per_tile = cols_per_tile * PER_COL
    nchunk = cols_per_tile // CHUNK_COLS
    npair = nchunk // 2

    @functools.partial(
        pl.kernel,
        out_type=jax.ShapeDtypeStruct((e_shard * NP,), jnp.float32),
        mesh=plsc.VectorSubcoreMesh(core_axis_name="c", subcore_axis_name="s"),
        compiler_params=pltpu.CompilerParams(use_tc_tiling_on_sc=False),
        scratch_types=[
            pltpu.VMEM((nnz_per_tile,), jnp.int32),
            pltpu.VMEM((nnz_per_tile,), jnp.float32),
            pltpu.VMEM((ROWS_PER_CHUNK, N), jnp.float32),
            pltpu.VMEM((ROWS_PER_CHUNK, N), jnp.float32),
            pltpu.VMEM((CHUNK_COLS * NP,), jnp.float32),
            pltpu.VMEM((CHUNK_COLS * NP,), jnp.float32),
            pltpu.SemaphoreType.DMA,
            pltpu.SemaphoreType.DMA,
            pltpu.SemaphoreType.DMA,
            pltpu.SemaphoreType.DMA,
        ],
    )
    def sc_body(xT, vals, ridx, out_o,
                idx_v, val_v, rows_a, rows_b, ob_a, ob_b,
                sem_a, sem_b, sem_oa, sem_ob):
        wid = lax.axis_index("s") * NC + lax.axis_index("c")
        nnz_base = shard * e_shard * PER_COL + wid * nnz_per_tile
        col_base = wid * cols_per_tile

        c1 = pltpu.async_copy(ridx.at[pl.ds(nnz_base, nnz_per_tile)], idx_v, sem_a)
        c2 = pltpu.async_copy(vals.at[pl.ds(nnz_base, nnz_per_tile)], val_v, sem_a)
        c1.wait()
        c2.wait()

        def issue_gather(g, buf, sem):
            for i in range(GPC):
                off = g * ROWS_PER_CHUNK + i * IDX_BLK
                pltpu.async_copy(
                    xT.at[idx_v.at[pl.ds(off, IDX_BLK)]],
                    buf.at[pl.ds(i * IDX_BLK, IDX_BLK)],
                    sem,
                )

        def wait_gather(buf, sem):
            for i in range(GPC):
                pltpu.make_async_copy(
                    xT.at[idx_v.at[pl.ds(i * IDX_BLK, IDX_BLK)]],
                    buf.at[pl.ds(i * IDX_BLK, IDX_BLK)],
                    sem,
                ).wait()

        def compute_chunk(g, buf, ob):
            @plsc.parallel_loop(0, CHUNK_COLS, unroll=1)
            def col_body(c):
                j0 = c * PER_COL
                jv = g * ROWS_PER_CHUNK + c * PER_COL
                vv = val_v[pl.ds(jv, PER_COL)]
                for l in range(NLB):
                    lanes = pl.ds(l * LANE, LANE)
                    xv = [buf[j0 + k, lanes] + vv[k] for k in range(PER_COL)]
                    m = _tree_max(xv)
                    s = _tree_sum([jnp.exp(v - m) for v in xv])
                    ob[pl.ds(c * NP + l * LANE, LANE)] = m
                    ob[pl.ds(c * NP + N + l * LANE, LANE)] = s

        def issue_out(g, ob, sem):
            dst = pl.ds((col_base + g * CHUNK_COLS) * NP, CHUNK_COLS * NP)
            pltpu.async_copy(ob, out_o.at[dst], sem)

        def wait_out(ob, sem):
            pltpu.make_async_copy(
                ob, out_o.at[pl.ds(0, CHUNK_COLS * NP)], sem).wait()

        issue_gather(0, rows_a, sem_a)

        def pair_body(p, carry):
            g0 = p * 2
            issue_gather(g0 + 1, rows_b, sem_b)
            wait_gather(rows_a, sem_a)

            @pl.when(p > 0)
            def _():
                wait_out(ob_a, sem_oa)

            compute_chunk(g0, rows_a, ob_a)
            issue_out(g0, ob_a, sem_oa)

            @pl.when(p + 1 < npair)
            def _():
                issue_gather(g0 + 2, rows_a, sem_a)

            wait_gather(rows_b, sem_b)

            @pl.when(p > 0)
            def _():
                wait_out(ob_b, sem_ob)

            compute_chunk(g0 + 1, rows_b, ob_b)
            issue_out(g0 + 1, ob_b, sem_ob)
            return carry

        lax.fori_loop(0, npair, pair_body, 0)
        wait_out(ob_a, sem_oa)
        wait_out(ob_b, sem_ob)

    return sc_body


_N_SHARD = 1
_sc_shards = [_make_sc(_N_SHARD, s) for s in range(_N_SHARD)]


_EB = 4096


def _finish_body(packed_ref, o_ref):
    mb = packed_ref[...].reshape(_EB, NP)
    o_ref[...] = (mb[:, :N] + jnp.log(mb[:, N:])).T


def _make_finish(e_shard):
    return pl.pallas_call(
        _finish_body,
        grid=(e_shard // _EB,),
        in_specs=[pl.BlockSpec((_EB * NP,), lambda i: (i,))],
        out_specs=pl.BlockSpec((N, _EB), lambda i: (0, i)),
        out_shape=jax.ShapeDtypeStruct((N, e_shard), jnp.float32),
    )


_finish = _make_finish(E // _N_SHARD)


def kernel(x, A_values, A_row_idx, A_col_idx):
    del A_col_idx
    xT = x.T
    ridx = A_row_idx.astype(jnp.int32)
    parts = []
    for sc in _sc_shards:
        packed = sc(xT, A_values, ridx)
        parts.append(_finish(packed))
    return jnp.concatenate(parts, axis=1) if len(parts) > 1 else parts[0]


# speedup vs baseline: 1.1856x; 1.0035x over previous
import functools

import jax
import jax.numpy as jnp
from jax import lax
from jax.experimental import pallas as pl
from jax.experimental.pallas import tpu as pltpu
from jax.experimental.pallas import tpu_sc as plsc

N = 64
D = 16384
E = 16384
PER_COL = 16

NC = 2
NS = 16
NW = NC * NS

NP = 128
CHUNK_COLS = 16
ROWS_PER_CHUNK = CHUNK_COLS * PER_COL
IDX_BLK = 128
GPC = ROWS_PER_CHUNK // IDX_BLK

LANE = 16
NLB = N // LANE


def _tree_max(vs):
    while len(vs) > 1:
        vs = [jnp.maximum(vs[i], vs[i + 1]) for i in range(0, len(vs) - 1, 2)] + (
            [vs[-1]] if len(vs) % 2 else [])
    return vs[0]


def _tree_sum(vs):
    while len(vs) > 1:
        vs = [vs[i] + vs[i + 1] for i in range(0, len(vs) - 1, 2)] + (
            [vs[-1]] if len(vs) % 2 else [])
    return vs[0]


def _make_sc(n_shard, shard):
    e_shard = E // n_shard
    cols_per_tile = e_shard // NW
    nnz_per_tile = cols_per_tile * PER_COL
    nchunk = cols_per_tile // CHUNK_COLS
    npair = nchunk // 2

    @functools.partial(
        pl.kernel,
        out_type=jax.ShapeDtypeStruct((e_shard * NP,), jnp.float32),
        mesh=plsc.VectorSubcoreMesh(core_axis_name="c", subcore_axis_name="s"),
        compiler_params=pltpu.CompilerParams(use_tc_tiling_on_sc=False),
        scratch_types=[
            pltpu.VMEM((nnz_per_tile,), jnp.int32),
            pltpu.VMEM((nnz_per_tile,), jnp.float32),
            pltpu.VMEM((ROWS_PER_CHUNK, N), jnp.float32),
            pltpu.VMEM((ROWS_PER_CHUNK, N), jnp.float32),
            pltpu.VMEM((CHUNK_COLS * NP,), jnp.float32),
            pltpu.VMEM((CHUNK_COLS * NP,), jnp.float32),
            pltpu.SemaphoreType.DMA,
            pltpu.SemaphoreType.DMA,
            pltpu.SemaphoreType.DMA,
            pltpu.SemaphoreType.DMA,
        ],
    )
    def sc_body(xT, vals, ridx, out_o,
                idx_v, val_v, rows_a, rows_b, ob_a, ob_b,
                sem_a, sem_b, sem_oa, sem_ob):
        wid = lax.axis_index("s") * NC + lax.axis_index("c")
        nnz_base = shard * e_shard * PER_COL + wid * nnz_per_tile
        col_base = wid * cols_per_tile

        c1 = pltpu.async_copy(ridx.at[pl.ds(nnz_base, nnz_per_tile)], idx_v, sem_a)
        c2 = pltpu.async_copy(vals.at[pl.ds(nnz_base, nnz_per_tile)], val_v, sem_a)
        c1.wait()
        c2.wait()

        def issue_gather(g, buf, sem):
            for i in range(GPC):
                off = g * ROWS_PER_CHUNK + i * IDX_BLK
                pltpu.async_copy(
                    xT.at[idx_v.at[pl.ds(off, IDX_BLK)]],
                    buf.at[pl.ds(i * IDX_BLK, IDX_BLK)],
                    sem,
                )

        def wait_gather(buf, sem):
            for i in range(GPC):
                pltpu.make_async_copy(
                    xT.at[idx_v.at[pl.ds(i * IDX_BLK, IDX_BLK)]],
                    buf.at[pl.ds(i * IDX_BLK, IDX_BLK)],
                    sem,
                ).wait()

        def compute_chunk(g, buf, ob):
            @plsc.parallel_loop(0, CHUNK_COLS, unroll=1)
            def col_body(c):
                j0 = c * PER_COL
                jv = g * ROWS_PER_CHUNK + c * PER_COL
                vv = val_v[pl.ds(jv, PER_COL)]
                for l in range(NLB):
                    lanes = pl.ds(l * LANE, LANE)
                    xv = [buf[j0 + k, lanes] + vv[k] for k in range(PER_COL)]
                    m = _tree_max(xv)
                    s = _tree_sum([jnp.exp(v - m) for v in xv])
                    ob[pl.ds(c * NP + l * LANE, LANE)] = m
                    ob[pl.ds(c * NP + N + l * LANE, LANE)] = s

        def issue_out(g, ob, sem):
            dst = pl.ds((col_base + g * CHUNK_COLS) * NP, CHUNK_COLS * NP)
            pltpu.async_copy(ob, out_o.at[dst], sem)

        def wait_out(ob, sem):
            pltpu.make_async_copy(
                ob, out_o.at[pl.ds(0, CHUNK_COLS * NP)], sem).wait()

        issue_gather(0, rows_a, sem_a)

        def pair_body(p, carry):
            g0 = p * 2
            issue_gather(g0 + 1, rows_b, sem_b)
            wait_gather(rows_a, sem_a)

            @pl.when(p > 0)
            def _():
                wait_out(ob_a, sem_oa)

            compute_chunk(g0, rows_a, ob_a)
            issue_out(g0, ob_a, sem_oa)

            @pl.when(p + 1 < npair)
            def _():
                issue_gather(g0 + 2, rows_a, sem_a)

            wait_gather(rows_b, sem_b)

            @pl.when(p > 0)
            def _():
                wait_out(ob_b, sem_ob)

            compute_chunk(g0 + 1, rows_b, ob_b)
            issue_out(g0 + 1, ob_b, sem_ob)
            return carry

        lax.fori_loop(0, npair, pair_body, 0)
        wait_out(ob_a, sem_oa)
        wait_out(ob_b, sem_ob)

    return sc_body


_N_SHARD = 1
_sc_shards = [_make_sc(_N_SHARD, s) for s in range(_N_SHARD)]


_EB = 8192


def _finish_body(packed_ref, o_ref):
    mb = packed_ref[...].reshape(_EB, NP)
    o_ref[...] = (mb[:, :N] + jnp.log(mb[:, N:])).T


def _make_finish(e_shard):
    return pl.pallas_call(
        _finish_body,
        grid=(e_shard // _EB,),
        in_specs=[pl.BlockSpec((_EB * NP,), lambda i: (i,))],
        out_specs=pl.BlockSpec((N, _EB), lambda i: (0, i)),
        out_shape=jax.ShapeDtypeStruct((N, e_shard), jnp.float32),
    )


_finish = _make_finish(E // _N_SHARD)


def kernel(x, A_values, A_row_idx, A_col_idx):
    del A_col_idx
    xT = x.T
    ridx = A_row_idx.astype(jnp.int32)
    parts = []
    for sc in _sc_shards:
        packed = sc(xT, A_values, ridx)
        parts.append(_finish(packed))
    return jnp.concatenate(parts, axis=1) if len(parts) > 1 else parts[0]


# speedup vs baseline: 1.2372x; 1.0435x over previous
import functools

import jax
import jax.numpy as jnp
from jax import lax
from jax.experimental import pallas as pl
from jax.experimental.pallas import tpu as pltpu
from jax.experimental.pallas import tpu_sc as plsc

N = 64
D = 16384
E = 16384
PER_COL = 16

NC = 2
NS = 16
NW = NC * NS

NP = 128
CHUNK_COLS = 16
ROWS_PER_CHUNK = CHUNK_COLS * PER_COL
IDX_BLK = 128
GPC = ROWS_PER_CHUNK // IDX_BLK

LANE = 16
NLB = N // LANE


def _tree_max(vs):
    while len(vs) > 1:
        vs = [jnp.maximum(vs[i], vs[i + 1]) for i in range(0, len(vs) - 1, 2)] + (
            [vs[-1]] if len(vs) % 2 else [])
    return vs[0]


def _tree_sum(vs):
    while len(vs) > 1:
        vs = [vs[i] + vs[i + 1] for i in range(0, len(vs) - 1, 2)] + (
            [vs[-1]] if len(vs) % 2 else [])
    return vs[0]


def _make_sc(n_shard, shard):
    e_shard = E // n_shard
    cols_per_tile = e_shard // NW
    nnz_per_tile = cols_per_tile * PER_COL
    nchunk = cols_per_tile // CHUNK_COLS
    npair = nchunk // 2

    @functools.partial(
        pl.kernel,
        out_type=jax.ShapeDtypeStruct((e_shard * NP,), jnp.float32),
        mesh=plsc.VectorSubcoreMesh(core_axis_name="c", subcore_axis_name="s"),
        compiler_params=pltpu.CompilerParams(use_tc_tiling_on_sc=False),
        scratch_types=[
            pltpu.VMEM((nnz_per_tile,), jnp.int32),
            pltpu.VMEM((nnz_per_tile,), jnp.float32),
            pltpu.VMEM((ROWS_PER_CHUNK, N), jnp.float32),
            pltpu.VMEM((ROWS_PER_CHUNK, N), jnp.float32),
            pltpu.VMEM((CHUNK_COLS * NP,), jnp.float32),
            pltpu.VMEM((CHUNK_COLS * NP,), jnp.float32),
            pltpu.SemaphoreType.DMA,
            pltpu.SemaphoreType.DMA,
            pltpu.SemaphoreType.DMA,
            pltpu.SemaphoreType.DMA,
        ],
    )
    def sc_body(xT, vals, ridx, out_o,
                idx_v, val_v, rows_a, rows_b, ob_a, ob_b,
                sem_a, sem_b, sem_oa, sem_ob):
        wid = lax.axis_index("s") * NC + lax.axis_index("c")
        nnz_base = shard * e_shard * PER_COL + wid * nnz_per_tile
        col_base = wid * cols_per_tile

        c1 = pltpu.async_copy(ridx.at[pl.ds(nnz_base, nnz_per_tile)], idx_v, sem_a)
        c2 = pltpu.async_copy(vals.at[pl.ds(nnz_base, nnz_per_tile)], val_v, sem_a)
        c1.wait()
        c2.wait()

        def issue_gather(g, buf, sem):
            for i in range(GPC):
                off = g * ROWS_PER_CHUNK + i * IDX_BLK
                pltpu.async_copy(
                    xT.at[idx_v.at[pl.ds(off, IDX_BLK)]],
                    buf.at[pl.ds(i * IDX_BLK, IDX_BLK)],
                    sem,
                )

        def wait_gather(buf, sem):
            for i in range(GPC):
                pltpu.make_async_copy(
                    xT.at[idx_v.at[pl.ds(i * IDX_BLK, IDX_BLK)]],
                    buf.at[pl.ds(i * IDX_BLK, IDX_BLK)],
                    sem,
                ).wait()

        def compute_chunk(g, buf, ob):
            @plsc.parallel_loop(0, CHUNK_COLS, unroll=1)
            def col_body(c):
                j0 = c * PER_COL
                jv = g * ROWS_PER_CHUNK + c * PER_COL
                vv = val_v[pl.ds(jv, PER_COL)]
                for l in range(NLB):
                    lanes = pl.ds(l * LANE, LANE)
                    xv = [buf[j0 + k, lanes] + vv[k] for k in range(PER_COL)]
                    m = _tree_max(xv)
                    s = _tree_sum([jnp.exp(v - m) for v in xv])
                    ob[pl.ds(c * NP + l * LANE, LANE)] = m
                    ob[pl.ds(c * NP + N + l * LANE, LANE)] = s

        def issue_out(g, ob, sem):
            dst = pl.ds((col_base + g * CHUNK_COLS) * NP, CHUNK_COLS * NP)
            pltpu.async_copy(ob, out_o.at[dst], sem)

        def wait_out(ob, sem):
            pltpu.make_async_copy(
                ob, out_o.at[pl.ds(0, CHUNK_COLS * NP)], sem).wait()

        issue_gather(0, rows_a, sem_a)

        def pair_body(p, carry):
            g0 = p * 2
            issue_gather(g0 + 1, rows_b, sem_b)
            wait_gather(rows_a, sem_a)

            @pl.when(p > 0)
            def _():
                wait_out(ob_a, sem_oa)

            compute_chunk(g0, rows_a, ob_a)
            issue_out(g0, ob_a, sem_oa)

            @pl.when(p + 1 < npair)
            def _():
                issue_gather(g0 + 2, rows_a, sem_a)

            wait_gather(rows_b, sem_b)

            @pl.when(p > 0)
            def _():
                wait_out(ob_b, sem_ob)

            compute_chunk(g0 + 1, rows_b, ob_b)
            issue_out(g0 + 1, ob_b, sem_ob)
            return carry

        lax.fori_loop(0, npair, pair_body, 0)
        wait_out(ob_a, sem_oa)
        wait_out(ob_b, sem_ob)

    return sc_body


_N_SHARD = 1
_sc_shards = [_make_sc(_N_SHARD, s) for s in range(_N_SHARD)]


_TB = 2048


def _xpose_body(x_ref, o_ref):
    t = x_ref[...].T.reshape(_TB // 2, 2, N)
    y = jnp.concatenate([t[:, 0, :], t[:, 1, :]], axis=1)
    o_ref[...] = y.reshape(_TB * N)


_xpose = pl.pallas_call(
    _xpose_body,
    grid=(D // _TB,),
    in_specs=[pl.BlockSpec((N, _TB), lambda i: (0, i))],
    out_specs=pl.BlockSpec((_TB * N,), lambda i: (i,)),
    out_shape=jax.ShapeDtypeStruct((D * N,), jnp.float32),
)


_EB = 8192


def _finish_body(packed_ref, o_ref):
    mb = packed_ref[...].reshape(_EB, NP)
    o_ref[...] = (mb[:, :N] + jnp.log(mb[:, N:])).T


def _make_finish(e_shard):
    return pl.pallas_call(
        _finish_body,
        grid=(e_shard // _EB,),
        in_specs=[pl.BlockSpec((_EB * NP,), lambda i: (i,))],
        out_specs=pl.BlockSpec((N, _EB), lambda i: (0, i)),
        out_shape=jax.ShapeDtypeStruct((N, e_shard), jnp.float32),
    )


_finish = _make_finish(E // _N_SHARD)


def kernel(x, A_values, A_row_idx, A_col_idx):
    del A_col_idx
    xT = _xpose(x).reshape(D, N)
    ridx = A_row_idx.astype(jnp.int32)
    parts = []
    for sc in _sc_shards:
        packed = sc(xT, A_values, ridx)
        parts.append(_finish(packed))
    return jnp.concatenate(parts, axis=1) if len(parts) > 1 else parts[0]
